# baseline (speedup 1.0000x reference)
import functools

import jax
import jax.numpy as jnp
from jax import lax
from jax.experimental import pallas as pl
from jax.experimental.pallas import tpu as pltpu
from jax.experimental.pallas import tpu_sc as plsc

N = 10000
E = 320000
D = 128
H = 32
OUT = 10
G = 128

NC = 2
NS = 16
NW = NC * NS
EW = E // NW
C = 128
NCH = 80
EWP = NCH * C
NB = 8
NBUF = NB
NPAD = N + 256
RZ = 624
RZTAIL = N - NS * RZ

PK = 4
NP = N // PK


def _sc_scatter_body(y_hbm, src_hbm, dst_hbm, zero_hbm, out0_hbm, out1_hbm,
                     acc, src_v, dst_v, rows, sem_g, sem_s):
    c = lax.axis_index("c")
    s = lax.axis_index("s")
    w = s * NC + c

    pltpu.sync_copy(zero_hbm.at[pl.ds(s * RZ, RZ)], acc.at[pl.ds(s * RZ, RZ)])

    @pl.when(s == NS - 1)
    def _():
        pltpu.sync_copy(zero_hbm.at[pl.ds(NS * RZ, RZTAIL)],
                        acc.at[pl.ds(NS * RZ, RZTAIL)])

    pltpu.sync_copy(src_hbm.at[w], src_v)
    pltpu.sync_copy(dst_hbm.at[w], dst_v)
    plsc.subcore_barrier()

    def gather(j, b):
        pltpu.async_copy(y_hbm.at[src_v.at[j]], rows[b], sem_g[b])

    def gather_wait(j, b):
        pltpu.make_async_copy(y_hbm.at[src_v.at[j]], rows[b], sem_g[b]).wait()

    def scat(j, b):
        pltpu.async_copy(rows[b], acc.at[dst_v.at[j]], sem_s[b], add=True)

    def scat_wait(j, b):
        pltpu.make_async_copy(rows[b], acc.at[dst_v.at[j]], sem_s[b]).wait()

    for b in range(NB):
        gather(b, b)

    @pl.loop(0, NCH - NB, step=NB)
    def _grp(o):
        for b in range(NB):
            j = o + b
            gather_wait(j, b)
            scat(j, b)
            scat_wait(j, b)
            gather(j + NB, b)

    for b in range(NB):
        j = NCH - NB + b
        gather_wait(j, b)
        scat(j, b)
        scat_wait(j, b)

    plsc.subcore_barrier()
    for ci, out_hbm in enumerate((out0_hbm, out1_hbm)):
        @pl.when(c == ci)
        def _():
            pltpu.sync_copy(acc.at[pl.ds(s * RZ, RZ)],
                            out_hbm.at[pl.ds(s * RZ, RZ)])

            @pl.when(s == NS - 1)
            def _():
                pltpu.sync_copy(acc.at[pl.ds(NS * RZ, RZTAIL)],
                                out_hbm.at[pl.ds(NS * RZ, RZTAIL)])


@functools.cache
def _make_sc_scatter():
    return pl.kernel(
        _sc_scatter_body,
        out_type=[jax.ShapeDtypeStruct((N, H), jnp.float32),
                  jax.ShapeDtypeStruct((N, H), jnp.float32)],
        mesh=plsc.VectorSubcoreMesh(core_axis_name="c", subcore_axis_name="s"),
        compiler_params=pltpu.CompilerParams(use_tc_tiling_on_sc=False),
        scratch_types=[
            pltpu.VMEM_SHARED((NPAD, H), jnp.float32),
            pltpu.VMEM((NCH, C), jnp.int32),
            pltpu.VMEM((NCH, C), jnp.int32),
            tuple(pltpu.VMEM((C, H), jnp.float32) for _ in range(NBUF)),
            tuple(pltpu.SemaphoreType.DMA for _ in range(NBUF)),
            tuple(pltpu.SemaphoreType.DMA for _ in range(NBUF)),
        ],
    )


def _proj_body(x_ref, wq_ref, y_ref):
    y4 = jnp.dot(x_ref[...], wq_ref[...], preferred_element_type=jnp.float32)
    t = y4.reshape(NP, PK, PK * H)
    y_ref[...] = jnp.concatenate(
        [t[:, k, k * H:(k + 1) * H] for k in range(PK)], axis=-1)


def _mlp1_body(y0_ref, agga_ref, aggb_ref, b01_ref, w02_ref, b02_ref,
               w11_ref, y1_ref):
    z = y0_ref[...] + agga_ref[...] + aggb_ref[...] + b01_ref[...]
    h = jnp.dot(jnp.maximum(z, 0.0), w02_ref[...],
                preferred_element_type=jnp.float32) + b02_ref[...]
    h = jnp.maximum(h, 0.0)
    y1_ref[...] = jnp.dot(h, w11_ref[...], preferred_element_type=jnp.float32)


def _mlp2_body(y1_ref, agga_ref, aggb_ref, b11_ref, w12_ref, b12_ref,
               batchq_ref, wp1_ref, bp1_ref, wp2_ref, bp2_ref,
               emb_ref, out_ref):
    z = y1_ref[...] + agga_ref[...] + aggb_ref[...] + b11_ref[...]
    emb_p = jnp.dot(jnp.maximum(z, 0.0), w12_ref[...],
                    preferred_element_type=jnp.float32) + b12_ref[...]
    emb_ref[...] = emb_p
    hr = jnp.maximum(emb_p, 0.0)

    bq = batchq_ref[...]
    seg = lax.broadcasted_iota(jnp.int32, (G, NP), 0)
    sums = jnp.zeros((G, H), jnp.float32)
    cnts = jnp.zeros((G, 1), jnp.float32)
    for k in range(PK):
        onehot = jnp.where(seg == bq[k][None, :], 1.0, 0.0)
        sums += jnp.dot(onehot, hr[:, k * H:(k + 1) * H],
                        preferred_element_type=jnp.float32)
        cnts += jnp.sum(onehot, axis=1, keepdims=True)

    pooled = sums / jnp.maximum(cnts, 1.0)
    o = jnp.dot(pooled, wp1_ref[...],
                preferred_element_type=jnp.float32) + bp1_ref[...]
    o = jnp.dot(o, wp2_ref[...],
                preferred_element_type=jnp.float32) + bp2_ref[...]
    m = jnp.max(o, axis=1, keepdims=True)
    e = o - m
    out_ref[...] = e - jnp.log(jnp.sum(jnp.exp(e), axis=1, keepdims=True))


_proj = pl.pallas_call(
    _proj_body,
    out_shape=jax.ShapeDtypeStruct((NP, PK * H), jnp.float32),
)

_mlp1 = pl.pallas_call(
    _mlp1_body,
    out_shape=jax.ShapeDtypeStruct((NP, PK * H), jnp.float32),
)

_mlp2 = pl.pallas_call(
    _mlp2_body,
    out_shape=[
        jax.ShapeDtypeStruct((NP, PK * H), jnp.float32),
        jax.ShapeDtypeStruct((G, OUT), jnp.float32),
    ],
)


def _bd(w):
    return jnp.kron(jnp.eye(PK, dtype=w.dtype), w)


def _bt(b):
    return jnp.tile(b, PK).reshape(1, PK * b.shape[0])


def kernel(x, edge_index, batch, W01, b01, W02, b02, W11, b11, W12, b12,
           Wp1, bp1, Wp2, bp2):
    e2 = edge_index.reshape(2, NW, EW)
    src = jnp.concatenate(
        [e2[0], jnp.zeros((NW, EWP - EW), jnp.int32)], axis=1
    ).reshape(NW, NCH, C)
    padvals = N + jnp.arange(EWP - EW, dtype=jnp.int32) % 256
    dst = jnp.concatenate(
        [e2[1], jnp.broadcast_to(padvals, (NW, EWP - EW))], axis=1
    ).reshape(NW, NCH, C)
    zeros = jnp.zeros((N, H), jnp.float32)
    batch_q = batch.reshape(NP, PK).T

    sc_scatter = _make_sc_scatter()
    y0p = _proj(x, jnp.concatenate([W01] * PK, axis=1))
    agg0a, agg0b = sc_scatter(y0p.reshape(N, H), src, dst, zeros)
    y1p = _mlp1(y0p, agg0a.reshape(NP, PK * H), agg0b.reshape(NP, PK * H),
                _bt(b01), _bd(W02), _bt(b02), _bd(W11))
    agg1a, agg1b = sc_scatter(y1p.reshape(N, H), src, dst, zeros)
    emb_p, out2 = _mlp2(y1p, agg1a.reshape(NP, PK * H),
                        agg1b.reshape(NP, PK * H), _bt(b11),
                        _bd(W12), _bt(b12), batch_q, Wp1, bp1.reshape(1, H),
                        Wp2, bp2.reshape(1, OUT))
    return (emb_p.reshape(N, H), out2)


# speedup vs baseline: 2.1175x; 2.1175x over previous
import functools

import jax
import jax.numpy as jnp
from jax import lax
from jax.experimental import pallas as pl
from jax.experimental.pallas import tpu as pltpu
from jax.experimental.pallas import tpu_sc as plsc

N = 10000
E = 320000
D = 128
H = 32
OUT = 10
G = 128

NC = 2
NS = 16
NW = NC * NS
EW = E // NW
C = 80
NCH = EW // C
NB = 5
NPAD = N
RZ = 624
RZTAIL = N - NS * RZ

PK = 4
NP = N // PK


def _sc_scatter_body(y_hbm, src_hbm, dst_hbm, zero_hbm, out0_hbm, out1_hbm,
                     acc, src_v, dst_v, rows, sem_g, sem_s):
    c = lax.axis_index("c")
    s = lax.axis_index("s")
    w = s * NC + c

    pltpu.sync_copy(zero_hbm.at[pl.ds(s * RZ, RZ)], acc.at[pl.ds(s * RZ, RZ)])

    @pl.when(s == NS - 1)
    def _():
        pltpu.sync_copy(zero_hbm.at[pl.ds(NS * RZ, RZTAIL)],
                        acc.at[pl.ds(NS * RZ, RZTAIL)])

    pltpu.sync_copy(src_hbm.at[w], src_v)
    pltpu.sync_copy(dst_hbm.at[w], dst_v)
    plsc.subcore_barrier()

    def gather(j, b):
        pltpu.async_copy(y_hbm.at[src_v.at[j]], rows[b], sem_g[b])

    def gather_wait(j, b):
        pltpu.make_async_copy(y_hbm.at[src_v.at[j]], rows[b], sem_g[b]).wait()

    def scat(j, b):
        pltpu.async_copy(rows[b], acc.at[dst_v.at[j]], sem_s[b], add=True)

    def scat_wait(j, b):
        pltpu.make_async_copy(rows[b], acc.at[dst_v.at[j]], sem_s[b]).wait()

    for b in range(NB):
        gather(b, b)

    @pl.loop(0, NCH - NB, step=NB)
    def _grp(o):
        for b in range(NB):
            j = o + b
            gather_wait(j, b)
            scat(j, b)
            scat_wait(j, b)
            gather(j + NB, b)

    for b in range(NB):
        j = NCH - NB + b
        gather_wait(j, b)
        scat(j, b)
        scat_wait(j, b)

    plsc.subcore_barrier()
    for ci, out_hbm in enumerate((out0_hbm, out1_hbm)):
        @pl.when(c == ci)
        def _():
            pltpu.sync_copy(acc.at[pl.ds(s * RZ, RZ)],
                            out_hbm.at[pl.ds(s * RZ, RZ)])

            @pl.when(s == NS - 1)
            def _():
                pltpu.sync_copy(acc.at[pl.ds(NS * RZ, RZTAIL)],
                                out_hbm.at[pl.ds(NS * RZ, RZTAIL)])


@functools.cache
def _make_sc_scatter():
    return pl.kernel(
        _sc_scatter_body,
        out_type=[jax.ShapeDtypeStruct((N, H), jnp.float32),
                  jax.ShapeDtypeStruct((N, H), jnp.float32)],
        mesh=plsc.VectorSubcoreMesh(core_axis_name="c", subcore_axis_name="s"),
        compiler_params=pltpu.CompilerParams(use_tc_tiling_on_sc=False),
        scratch_types=[
            pltpu.VMEM_SHARED((NPAD, H), jnp.float32),
            pltpu.VMEM((NCH, C), jnp.int32),
            pltpu.VMEM((NCH, C), jnp.int32),
            tuple(pltpu.VMEM((C, H), jnp.float32) for _ in range(NB)),
            tuple(pltpu.SemaphoreType.DMA for _ in range(NB)),
            tuple(pltpu.SemaphoreType.DMA for _ in range(NB)),
        ],
    )


def _proj_body(x_ref, wq_ref, y_ref):
    y4 = jnp.dot(x_ref[...], wq_ref[...], preferred_element_type=jnp.float32)
    t = y4.reshape(NP, PK, PK * H)
    y_ref[...] = jnp.concatenate(
        [t[:, k, k * H:(k + 1) * H] for k in range(PK)], axis=-1)


def _mlp1_body(y0_ref, agga_ref, aggb_ref, b01_ref, w02_ref, b02_ref,
               w11_ref, y1_ref):
    z = y0_ref[...] + agga_ref[...] + aggb_ref[...] + b01_ref[...]
    h = jnp.dot(jnp.maximum(z, 0.0), w02_ref[...],
                preferred_element_type=jnp.float32) + b02_ref[...]
    h = jnp.maximum(h, 0.0)
    y1_ref[...] = jnp.dot(h, w11_ref[...], preferred_element_type=jnp.float32)


def _mlp2_body(y1_ref, agga_ref, aggb_ref, b11_ref, w12_ref, b12_ref,
               batchq_ref, wp1_ref, bp1_ref, wp2_ref, bp2_ref,
               emb_ref, out_ref):
    z = y1_ref[...] + agga_ref[...] + aggb_ref[...] + b11_ref[...]
    emb_p = jnp.dot(jnp.maximum(z, 0.0), w12_ref[...],
                    preferred_element_type=jnp.float32) + b12_ref[...]
    emb_ref[...] = emb_p
    hr = jnp.maximum(emb_p, 0.0)

    bq = batchq_ref[...]
    seg = lax.broadcasted_iota(jnp.int32, (G, NP), 0)
    sums = jnp.zeros((G, H), jnp.float32)
    cnts = jnp.zeros((G, 1), jnp.float32)
    for k in range(PK):
        onehot = jnp.where(seg == bq[k][None, :], 1.0, 0.0)
        sums += jnp.dot(onehot, hr[:, k * H:(k + 1) * H],
                        preferred_element_type=jnp.float32)
        cnts += jnp.sum(onehot, axis=1, keepdims=True)

    pooled = sums / jnp.maximum(cnts, 1.0)
    o = jnp.dot(pooled, wp1_ref[...],
                preferred_element_type=jnp.float32) + bp1_ref[...]
    o = jnp.dot(o, wp2_ref[...],
                preferred_element_type=jnp.float32) + bp2_ref[...]
    m = jnp.max(o, axis=1, keepdims=True)
    e = o - m
    out_ref[...] = e - jnp.log(jnp.sum(jnp.exp(e), axis=1, keepdims=True))


_proj = pl.pallas_call(
    _proj_body,
    out_shape=jax.ShapeDtypeStruct((NP, PK * H), jnp.float32),
)

_mlp1 = pl.pallas_call(
    _mlp1_body,
    out_shape=jax.ShapeDtypeStruct((NP, PK * H), jnp.float32),
)

_mlp2 = pl.pallas_call(
    _mlp2_body,
    out_shape=[
        jax.ShapeDtypeStruct((NP, PK * H), jnp.float32),
        jax.ShapeDtypeStruct((G, OUT), jnp.float32),
    ],
)


def _bd(w):
    return jnp.kron(jnp.eye(PK, dtype=w.dtype), w)


def _bt(b):
    return jnp.tile(b, PK).reshape(1, PK * b.shape[0])


def kernel(x, edge_index, batch, W01, b01, W02, b02, W11, b11, W12, b12,
           Wp1, bp1, Wp2, bp2):
    src = edge_index[0].reshape(NW, NCH, C)
    dst = edge_index[1].reshape(NW, NCH, C)
    zeros = jnp.zeros((N, H), jnp.float32)
    batch_q = batch.reshape(NP, PK).T

    sc_scatter = _make_sc_scatter()
    y0p = _proj(x, jnp.concatenate([W01] * PK, axis=1))
    agg0a, agg0b = sc_scatter(y0p.reshape(N, H), src, dst, zeros)
    y1p = _mlp1(y0p, agg0a.reshape(NP, PK * H), agg0b.reshape(NP, PK * H),
                _bt(b01), _bd(W02), _bt(b02), _bd(W11))
    agg1a, agg1b = sc_scatter(y1p.reshape(N, H), src, dst, zeros)
    emb_p, out2 = _mlp2(y1p, agg1a.reshape(NP, PK * H),
                        agg1b.reshape(NP, PK * H), _bt(b11),
                        _bd(W12), _bt(b12), batch_q, Wp1, bp1.reshape(1, H),
                        Wp2, bp2.reshape(1, OUT))
    return (emb_p.reshape(N, H), out2)


# speedup vs baseline: 2.1234x; 1.0028x over previous
import functools

import jax
import jax.numpy as jnp
from jax import lax
from jax.experimental import pallas as pl
from jax.experimental.pallas import tpu as pltpu
from jax.experimental.pallas import tpu_sc as plsc

N = 10000
E = 320000
D = 128
H = 32
OUT = 10
G = 128

NC = 2
NS = 16
NW = NC * NS
EW = E // NW
C = 80
NCH = EW // C
NB = 5
NBUF = 2 * NB
NPAD = N
RZ = 624
RZTAIL = N - NS * RZ

PK = 4
NP = N // PK


def _sc_scatter_body(y_hbm, src_hbm, dst_hbm, zero_hbm, out0_hbm, out1_hbm,
                     acc, src_v, dst_v, rows, sem_g, sem_s):
    c = lax.axis_index("c")
    s = lax.axis_index("s")
    w = s * NC + c

    pltpu.sync_copy(zero_hbm.at[pl.ds(s * RZ, RZ)], acc.at[pl.ds(s * RZ, RZ)])

    @pl.when(s == NS - 1)
    def _():
        pltpu.sync_copy(zero_hbm.at[pl.ds(NS * RZ, RZTAIL)],
                        acc.at[pl.ds(NS * RZ, RZTAIL)])

    pltpu.sync_copy(src_hbm.at[w], src_v)
    pltpu.sync_copy(dst_hbm.at[w], dst_v)
    plsc.subcore_barrier()

    def gather(j, b):
        pltpu.async_copy(y_hbm.at[src_v.at[j]], rows[b], sem_g[b])

    def gather_wait(j, b):
        pltpu.make_async_copy(y_hbm.at[src_v.at[j]], rows[b], sem_g[b]).wait()

    def scat(j, b):
        pltpu.async_copy(rows[b], acc.at[dst_v.at[j]], sem_s[b], add=True)

    def scat_wait(j, b):
        pltpu.make_async_copy(rows[b], acc.at[dst_v.at[j]], sem_s[b]).wait()

    NMAIN = ((NCH - 2 * NB) // NBUF) * NBUF

    for j in range(NB):
        gather(j, j)
    for j in range(NB):
        gather_wait(j, j)
        scat(j, j)
        gather(j + NB, j + NB)

    @pl.loop(NB, NB + NMAIN, step=NBUF)
    def _grp(o):
        for t in range(NBUF):
            j = o + t
            b = (NB + t) % NBUF
            gather_wait(j, b)
            scat(j, b)
            scat_wait(j - NB, (b + NB) % NBUF)
            gather(j + NB, (b + NB) % NBUF)

    for t in range(NCH - 2 * NB - NMAIN):
        j = NB + NMAIN + t
        b = j % NBUF
        gather_wait(j, b)
        scat(j, b)
        scat_wait(j - NB, (b + NB) % NBUF)
        gather(j + NB, (b + NB) % NBUF)

    for t in range(NB):
        j = NCH - NB + t
        b = j % NBUF
        gather_wait(j, b)
        scat(j, b)

    for j in range(NCH - NBUF, NCH):
        scat_wait(j, j % NBUF)

    plsc.subcore_barrier()
    for ci, out_hbm in enumerate((out0_hbm, out1_hbm)):
        @pl.when(c == ci)
        def _():
            pltpu.sync_copy(acc.at[pl.ds(s * RZ, RZ)],
                            out_hbm.at[pl.ds(s * RZ, RZ)])

            @pl.when(s == NS - 1)
            def _():
                pltpu.sync_copy(acc.at[pl.ds(NS * RZ, RZTAIL)],
                                out_hbm.at[pl.ds(NS * RZ, RZTAIL)])


@functools.cache
def _make_sc_scatter():
    return pl.kernel(
        _sc_scatter_body,
        out_type=[jax.ShapeDtypeStruct((N, H), jnp.float32),
                  jax.ShapeDtypeStruct((N, H), jnp.float32)],
        mesh=plsc.VectorSubcoreMesh(core_axis_name="c", subcore_axis_name="s"),
        compiler_params=pltpu.CompilerParams(use_tc_tiling_on_sc=False),
        scratch_types=[
            pltpu.VMEM_SHARED((NPAD, H), jnp.float32),
            pltpu.VMEM((NCH, C), jnp.int32),
            pltpu.VMEM((NCH, C), jnp.int32),
            tuple(pltpu.VMEM((C, H), jnp.float32) for _ in range(NBUF)),
            tuple(pltpu.SemaphoreType.DMA for _ in range(NBUF)),
            tuple(pltpu.SemaphoreType.DMA for _ in range(NBUF)),
        ],
    )


def _proj_body(x_ref, wq_ref, y_ref):
    y4 = jnp.dot(x_ref[...], wq_ref[...], preferred_element_type=jnp.float32)
    t = y4.reshape(NP, PK, PK * H)
    y_ref[...] = jnp.concatenate(
        [t[:, k, k * H:(k + 1) * H] for k in range(PK)], axis=-1)


def _mlp1_body(y0_ref, agga_ref, aggb_ref, b01_ref, w02_ref, b02_ref,
               w11_ref, y1_ref):
    z = y0_ref[...] + agga_ref[...] + aggb_ref[...] + b01_ref[...]
    h = jnp.dot(jnp.maximum(z, 0.0), w02_ref[...],
                preferred_element_type=jnp.float32) + b02_ref[...]
    h = jnp.maximum(h, 0.0)
    y1_ref[...] = jnp.dot(h, w11_ref[...], preferred_element_type=jnp.float32)


def _mlp2_body(y1_ref, agga_ref, aggb_ref, b11_ref, w12_ref, b12_ref,
               batchq_ref, wp1_ref, bp1_ref, wp2_ref, bp2_ref,
               emb_ref, out_ref):
    z = y1_ref[...] + agga_ref[...] + aggb_ref[...] + b11_ref[...]
    emb_p = jnp.dot(jnp.maximum(z, 0.0), w12_ref[...],
                    preferred_element_type=jnp.float32) + b12_ref[...]
    emb_ref[...] = emb_p
    hr = jnp.maximum(emb_p, 0.0)

    bq = batchq_ref[...]
    seg = lax.broadcasted_iota(jnp.int32, (G, NP), 0)
    sums = jnp.zeros((G, H), jnp.float32)
    cnts = jnp.zeros((G, 1), jnp.float32)
    for k in range(PK):
        onehot = jnp.where(seg == bq[k][None, :], 1.0, 0.0)
        sums += jnp.dot(onehot, hr[:, k * H:(k + 1) * H],
                        preferred_element_type=jnp.float32)
        cnts += jnp.sum(onehot, axis=1, keepdims=True)

    pooled = sums / jnp.maximum(cnts, 1.0)
    o = jnp.dot(pooled, wp1_ref[...],
                preferred_element_type=jnp.float32) + bp1_ref[...]
    o = jnp.dot(o, wp2_ref[...],
                preferred_element_type=jnp.float32) + bp2_ref[...]
    m = jnp.max(o, axis=1, keepdims=True)
    e = o - m
    out_ref[...] = e - jnp.log(jnp.sum(jnp.exp(e), axis=1, keepdims=True))


_proj = pl.pallas_call(
    _proj_body,
    out_shape=jax.ShapeDtypeStruct((NP, PK * H), jnp.float32),
)

_mlp1 = pl.pallas_call(
    _mlp1_body,
    out_shape=jax.ShapeDtypeStruct((NP, PK * H), jnp.float32),
)

_mlp2 = pl.pallas_call(
    _mlp2_body,
    out_shape=[
        jax.ShapeDtypeStruct((NP, PK * H), jnp.float32),
        jax.ShapeDtypeStruct((G, OUT), jnp.float32),
    ],
)


def _bd(w):
    return jnp.kron(jnp.eye(PK, dtype=w.dtype), w)


def _bt(b):
    return jnp.tile(b, PK).reshape(1, PK * b.shape[0])


def kernel(x, edge_index, batch, W01, b01, W02, b02, W11, b11, W12, b12,
           Wp1, bp1, Wp2, bp2):
    src = edge_index[0].reshape(NW, NCH, C)
    dst = edge_index[1].reshape(NW, NCH, C)
    zeros = jnp.zeros((N, H), jnp.float32)
    batch_q = batch.reshape(NP, PK).T

    sc_scatter = _make_sc_scatter()
    y0p = _proj(x, jnp.concatenate([W01] * PK, axis=1))
    agg0a, agg0b = sc_scatter(y0p.reshape(N, H), src, dst, zeros)
    y1p = _mlp1(y0p, agg0a.reshape(NP, PK * H), agg0b.reshape(NP, PK * H),
                _bt(b01), _bd(W02), _bt(b02), _bd(W11))
    agg1a, agg1b = sc_scatter(y1p.reshape(N, H), src, dst, zeros)
    emb_p, out2 = _mlp2(y1p, agg1a.reshape(NP, PK * H),
                        agg1b.reshape(NP, PK * H), _bt(b11),
                        _bd(W12), _bt(b12), batch_q, Wp1, bp1.reshape(1, H),
                        Wp2, bp2.reshape(1, OUT))
    return (emb_p.reshape(N, H), out2)
